# baseline (speedup 1.0000x reference)
import functools

import jax
import jax.numpy as jnp
from jax import lax
from jax.experimental import pallas as pl
from jax.experimental.pallas import tpu as pltpu
from jax.experimental.pallas import tpu_sc as plsc

N = 10000
E = 160000
NC = 2
NS = 16
LANES = 16
BATCH = 128
NB = 40
E_PAD = NC * NS * NB * BATCH
N_PAD = 10240
ROWS_PER_SUB = N_PAD // NS
ZROWS = 128

_MESH = plsc.VectorSubcoreMesh(
    core_axis_name="c", subcore_axis_name="s", num_cores=NC, num_subcores=NS
)
_SC_PARAMS = pltpu.CompilerParams(needs_layout_passes=False)


def _deg_body(src_hbm, dst_hbm, val_hbm, out_o0, out_o1, out_i0, out_i1,
              src_v, dst_v, val_v, zero_v, acc_o, acc_i):
    c = lax.axis_index("c")
    s = lax.axis_index("s")
    pltpu.sync_copy(src_hbm.at[c, s], src_v)
    pltpu.sync_copy(dst_hbm.at[c, s], dst_v)
    pltpu.sync_copy(val_hbm.at[c, s], val_v)

    z16 = jnp.zeros((LANES,), jnp.float32)

    def zbody(k, carry):
        zero_v[pl.ds(k * LANES, LANES)] = z16
        return carry

    lax.fori_loop(0, 640 // LANES, zbody, 0)
    stripe = pl.ds(s * 640, 640)
    pltpu.sync_copy(zero_v, acc_o.at[stripe])
    pltpu.sync_copy(zero_v, acc_i.at[stripe])
    plsc.subcore_barrier()

    def body(b, carry):
        pltpu.sync_copy(val_v.at[b], acc_o.at[src_v.at[b]], add=True)
        pltpu.sync_copy(val_v.at[b], acc_i.at[dst_v.at[b]], add=True)
        return carry

    lax.fori_loop(0, NB, body, 0)
    plsc.subcore_barrier()

    @pl.when(c == 0)
    def _():
        pltpu.sync_copy(acc_o.at[stripe], out_o0.at[stripe])
        pltpu.sync_copy(acc_i.at[stripe], out_i0.at[stripe])

    @pl.when(c == 1)
    def _():
        pltpu.sync_copy(acc_o.at[stripe], out_o1.at[stripe])
        pltpu.sync_copy(acc_i.at[stripe], out_i1.at[stripe])


_deg = pl.kernel(
    _deg_body,
    out_type=tuple(jax.ShapeDtypeStruct((N_PAD,), jnp.float32) for _ in range(4)),
    mesh=_MESH,
    compiler_params=_SC_PARAMS,
    scratch_types=[
        pltpu.VMEM((NB, BATCH), jnp.int32),
        pltpu.VMEM((NB, BATCH), jnp.int32),
        pltpu.VMEM((NB, BATCH), jnp.float32),
        pltpu.VMEM((640,), jnp.float32),
        pltpu.VMEM_SHARED((N_PAD,), jnp.float32),
        pltpu.VMEM_SHARED((N_PAD,), jnp.float32),
    ],
)


def _coef_body(src_hbm, dst_hbm, ew_hbm, so_hbm, si_hbm, c_out,
               src_v, dst_v, ew_v, c_v, so_v, si_v):
    c = lax.axis_index("c")
    s = lax.axis_index("s")
    pltpu.sync_copy(src_hbm.at[c, s], src_v)
    pltpu.sync_copy(dst_hbm.at[c, s], dst_v)
    pltpu.sync_copy(ew_hbm.at[c, s], ew_v)
    pltpu.sync_copy(so_hbm, so_v)
    pltpu.sync_copy(si_hbm, si_v)

    def cbody(b, carry):
        for kk in range(BATCH // LANES):
            sl = pl.ds(kk * LANES, LANES)
            sv = plsc.load_gather(so_v, [src_v[b, sl]])
            dv = plsc.load_gather(si_v, [dst_v[b, sl]])
            c_v[b, sl] = ew_v[b, sl] * sv * dv
        return carry

    lax.fori_loop(0, NB, cbody, 0)
    pltpu.sync_copy(c_v, c_out.at[c, s])


_coef = pl.kernel(
    _coef_body,
    out_type=jax.ShapeDtypeStruct((NC, NS, NB, BATCH), jnp.float32),
    mesh=_MESH,
    compiler_params=_SC_PARAMS,
    scratch_types=[
        pltpu.VMEM((NB, BATCH), jnp.int32),
        pltpu.VMEM((NB, BATCH), jnp.int32),
        pltpu.VMEM((NB, BATCH), jnp.float32),
        pltpu.VMEM((NB, BATCH), jnp.float32),
        pltpu.VMEM((N_PAD,), jnp.float32),
        pltpu.VMEM((N_PAD,), jnp.float32),
    ],
)


def _bcast_lane(vec16, j):
    return lax.gather(
        vec16,
        jnp.full((LANES, 1), j, jnp.int32),
        lax.GatherDimensionNumbers(
            offset_dims=(), collapsed_slice_dims=(0,), start_index_map=(0,)
        ),
        (1,),
        mode=lax.GatherScatterMode.PROMISE_IN_BOUNDS,
    )


NB0 = 40
NB1 = 40
CHUNK = 40
TOTB = E_PAD // BATCH


def _spmm_body(d, x_hbm, src_hbm, dst_hbm, c_hbm, out_hbm,
               src_v, dst_v, c_v, rows0, rows1, acc,
               gsem0, gsem1, ssem0, ssem1):
    nd = d // LANES
    c = lax.axis_index("c")
    s = lax.axis_index("s")

    z16 = jnp.zeros((LANES,), jnp.float32)

    def zrow(i, carry):
        for kk in range(nd):
            rows0[i, pl.ds(kk * LANES, LANES)] = z16
        return carry

    lax.fori_loop(0, ZROWS, zrow, 0)
    for j in range(ROWS_PER_SUB // ZROWS):
        pltpu.sync_copy(rows0, acc.at[pl.ds(s * ROWS_PER_SUB + j * ZROWS, ZROWS)])
    plsc.subcore_barrier()

    def _gather(t, buf, sem):
        pltpu.async_copy(x_hbm.at[src_v.at[t]], buf, sem)

    def _wait_gather(t, buf, sem):
        pltpu.make_async_copy(x_hbm.at[src_v.at[t]], buf, sem).wait()

    def _mult(t, buf):
        @plsc.parallel_loop(0, BATCH // LANES, 1, unroll=4)
        def blk(k):
            w16 = c_v[t, pl.ds(k * LANES, LANES)]
            for j in range(LANES):
                w = _bcast_lane(w16, j)
                e = k * LANES + j
                for kk in range(nd):
                    sl = pl.ds(kk * LANES, LANES)
                    buf[e, sl] = buf[e, sl] * w

    def _run_chunk(base_b, nbc):
        sl_in = pl.ds(pl.multiple_of(base_b, 8), nbc)
        sl_v = pl.ds(0, nbc)
        pltpu.sync_copy(src_hbm.at[sl_in], src_v.at[sl_v])
        pltpu.sync_copy(dst_hbm.at[sl_in], dst_v.at[sl_v])
        pltpu.sync_copy(c_hbm.at[sl_in], c_v.at[sl_v])

        _gather(0, rows0, gsem0)
        _gather(1, rows1, gsem1)

        def mbody(k, carry):
            t0 = 2 * k
            t1 = 2 * k + 1
            _wait_gather(t0, rows0, gsem0)
            _mult(t0, rows0)
            pltpu.sync_copy(rows0, acc.at[dst_v.at[t0]], add=True)

            @pl.when(k < nbc // 2 - 1)
            def _():
                _gather(t0 + 2, rows0, gsem0)

            _wait_gather(t1, rows1, gsem1)
            _mult(t1, rows1)
            pltpu.sync_copy(rows1, acc.at[dst_v.at[t1]], add=True)

            @pl.when(k < nbc // 2 - 1)
            def _():
                _gather(t1 + 2, rows1, gsem1)

            return carry

        lax.fori_loop(0, nbc // 2, mbody, 0)

    @pl.when(c == 0)
    def _():
        _run_chunk(s * NB0, NB0)

    @pl.when(c == 1)
    def _():
        _run_chunk(NS * NB0 + s * NB1, NB1)

    plsc.subcore_barrier()
    for j in range(ROWS_PER_SUB // ZROWS):
        sl = pl.ds(s * ROWS_PER_SUB + j * ZROWS, ZROWS)
        pltpu.sync_copy(acc.at[sl], out_hbm.at[c, sl])


def _make_spmm(d):
    return pl.kernel(
        functools.partial(_spmm_body, d),
        out_type=jax.ShapeDtypeStruct((NC, N_PAD, d), jnp.float32),
        mesh=_MESH,
        compiler_params=_SC_PARAMS,
        scratch_types=[
            pltpu.VMEM((CHUNK, BATCH), jnp.int32),
            pltpu.VMEM((CHUNK, BATCH), jnp.int32),
            pltpu.VMEM((CHUNK, BATCH), jnp.float32),
            pltpu.VMEM((BATCH, d), jnp.float32),
            pltpu.VMEM((BATCH, d), jnp.float32),
            pltpu.VMEM_SHARED((N_PAD, d), jnp.float32),
            pltpu.SemaphoreType.DMA,
            pltpu.SemaphoreType.DMA,
            pltpu.SemaphoreType.DMA,
            pltpu.SemaphoreType.DMA,
        ],
    )


_spmm128 = _make_spmm(128)


def _scales_body(o0_ref, o1_ref, i0_ref, i1_ref, so_ref, si_ref):
    dego = o0_ref[...] + o1_ref[...]
    degi = i0_ref[...] + i1_ref[...]
    so_ref[...] = lax.rsqrt(jnp.maximum(dego, 1.0))
    si_ref[...] = lax.rsqrt(jnp.maximum(degi, 1.0))


_scales = pl.pallas_call(
    _scales_body,
    out_shape=(
        jax.ShapeDtypeStruct((N_PAD,), jnp.float32),
        jax.ShapeDtypeStruct((N_PAD,), jnp.float32),
    ),
)

_ROWB = 2000
_GRID = N // _ROWB


def _mm_body(x_ref, w_ref, o_ref):
    o_ref[...] = jnp.dot(x_ref[...], w_ref[...], preferred_element_type=jnp.float32)


def _psum_bias_mm_body(z_ref, b_ref, w_ref, o_ref):
    h = z_ref[0] + z_ref[1] + b_ref[...]
    o_ref[...] = jnp.dot(h, w_ref[...], preferred_element_type=jnp.float32)


def _psum_bias_body(z_ref, b_ref, o_ref):
    o_ref[...] = z_ref[0] + z_ref[1] + b_ref[...]


def _psum_mm_bias_body(z_ref, w_ref, b_ref, o_ref):
    h = z_ref[0] + z_ref[1]
    o_ref[...] = jnp.dot(h, w_ref[...], preferred_element_type=jnp.float32) + b_ref[...]


def _x_spec(dcol):
    return pl.BlockSpec((_ROWB, dcol), lambda i: (i, 0))


def _z_spec(dcol):
    return pl.BlockSpec((NC, _ROWB, dcol), lambda i: (0, i, 0))


def _w_spec(din, dout):
    return pl.BlockSpec((din, dout), lambda i: (0, 0))


def _b_spec(dout):
    return pl.BlockSpec((1, dout), lambda i: (0, 0))


def _out_shape(dout):
    return jax.ShapeDtypeStruct((N, dout), jnp.float32)


_stage_a = pl.pallas_call(
    _mm_body, grid=(_GRID,),
    in_specs=[_x_spec(256), _w_spec(256, 128)],
    out_specs=_x_spec(128), out_shape=_out_shape(128),
)

_stage_b = pl.pallas_call(
    _psum_bias_mm_body, grid=(_GRID,),
    in_specs=[_z_spec(128), _b_spec(128), _w_spec(128, 128)],
    out_specs=_x_spec(128), out_shape=_out_shape(128),
)

_stage_c = pl.pallas_call(
    _psum_bias_body, grid=(_GRID,),
    in_specs=[_z_spec(128), _b_spec(128)],
    out_specs=_x_spec(128), out_shape=_out_shape(128),
)

_stage_d = pl.pallas_call(
    _psum_mm_bias_body, grid=(_GRID,),
    in_specs=[_z_spec(128), _w_spec(128, 128), _b_spec(128)],
    out_specs=_x_spec(128), out_shape=_out_shape(128),
)

_stage_e = pl.pallas_call(
    _psum_mm_bias_body, grid=(_GRID,),
    in_specs=[_z_spec(128), _w_spec(128, 256), _b_spec(256)],
    out_specs=_x_spec(256), out_shape=_out_shape(256),
)


def kernel(feat, edge_index, edge_weight, W0, b0, W1, b1, W2, b2, W3, b3):
    pad = E_PAD - E
    src = edge_index[0].astype(jnp.int32)
    dst = edge_index[1].astype(jnp.int32)
    pi = lax.iota(jnp.int32, pad)
    src_pad = pi % N
    dst_pad = N + pi % (N_PAD - N)
    zf = jnp.zeros((pad,), jnp.float32)
    shape4 = (NC, NS, NB, BATCH)
    srcp = jnp.concatenate([src, src_pad]).reshape(shape4)
    dstp = jnp.concatenate([dst, dst_pad]).reshape(shape4)
    ewp = jnp.concatenate([edge_weight.astype(jnp.float32), zf]).reshape(shape4)
    valp = jnp.concatenate([jnp.ones((E,), jnp.float32), zf]).reshape(shape4)

    deg_o0, deg_o1, deg_i0, deg_i1 = _deg(srcp, dstp, valp)
    s_out, s_in = _scales(deg_o0, deg_o1, deg_i0, deg_i1)
    cedge = _coef(srcp, dstp, ewp, s_out, s_in)

    srcf = srcp.reshape(TOTB, BATCH)
    dstf = dstp.reshape(TOTB, BATCH)
    cf = cedge.reshape(TOTB, BATCH)

    w1p = jnp.concatenate([W1, jnp.zeros((128, 64), jnp.float32)], axis=1)
    b1p = jnp.concatenate([b1, jnp.zeros((64,), jnp.float32)])
    w2p = jnp.concatenate([W2, jnp.zeros((64, 128), jnp.float32)], axis=0)

    x0 = _stage_a(feat, W0)
    z0 = _spmm128(x0, srcf, dstf, cf)
    x1 = _stage_b(z0, b0.reshape(1, -1), w1p)
    z1 = _spmm128(x1, srcf, dstf, cf)
    x2 = _stage_c(z1, b1p.reshape(1, -1))
    z2 = _spmm128(x2, srcf, dstf, cf)
    x3 = _stage_d(z2, w2p, b2.reshape(1, -1))
    z3 = _spmm128(x3, srcf, dstf, cf)
    return _stage_e(z3, W3, b3.reshape(1, -1))


# speedup vs baseline: 1.0624x; 1.0624x over previous
import functools

import jax
import jax.numpy as jnp
from jax import lax
from jax.experimental import pallas as pl
from jax.experimental.pallas import tpu as pltpu
from jax.experimental.pallas import tpu_sc as plsc

N = 10000
E = 160000
NC = 2
NS = 16
LANES = 16
BATCH = 128
NB = 40
E_PAD = NC * NS * NB * BATCH
N_PAD = 10240
ROWS_PER_SUB = N_PAD // NS
ZROWS = 128

_MESH = plsc.VectorSubcoreMesh(
    core_axis_name="c", subcore_axis_name="s", num_cores=NC, num_subcores=NS
)
_SC_PARAMS = pltpu.CompilerParams(needs_layout_passes=False)


def _deg_body(src_hbm, dst_hbm, val_hbm, out_o0, out_o1, out_i0, out_i1,
              src_v, dst_v, val_v, zero_v, acc_o, acc_i):
    c = lax.axis_index("c")
    s = lax.axis_index("s")
    pltpu.sync_copy(src_hbm.at[c, s], src_v)
    pltpu.sync_copy(dst_hbm.at[c, s], dst_v)
    pltpu.sync_copy(val_hbm.at[c, s], val_v)

    z16 = jnp.zeros((LANES,), jnp.float32)

    def zbody(k, carry):
        zero_v[pl.ds(k * LANES, LANES)] = z16
        return carry

    lax.fori_loop(0, 640 // LANES, zbody, 0)
    stripe = pl.ds(s * 640, 640)
    pltpu.sync_copy(zero_v, acc_o.at[stripe])
    pltpu.sync_copy(zero_v, acc_i.at[stripe])
    plsc.subcore_barrier()

    def body(b, carry):
        pltpu.sync_copy(val_v.at[b], acc_o.at[src_v.at[b]], add=True)
        pltpu.sync_copy(val_v.at[b], acc_i.at[dst_v.at[b]], add=True)
        return carry

    lax.fori_loop(0, NB, body, 0)
    plsc.subcore_barrier()

    @pl.when(c == 0)
    def _():
        pltpu.sync_copy(acc_o.at[stripe], out_o0.at[stripe])
        pltpu.sync_copy(acc_i.at[stripe], out_i0.at[stripe])

    @pl.when(c == 1)
    def _():
        pltpu.sync_copy(acc_o.at[stripe], out_o1.at[stripe])
        pltpu.sync_copy(acc_i.at[stripe], out_i1.at[stripe])


_deg = pl.kernel(
    _deg_body,
    out_type=tuple(jax.ShapeDtypeStruct((N_PAD,), jnp.float32) for _ in range(4)),
    mesh=_MESH,
    compiler_params=_SC_PARAMS,
    scratch_types=[
        pltpu.VMEM((NB, BATCH), jnp.int32),
        pltpu.VMEM((NB, BATCH), jnp.int32),
        pltpu.VMEM((NB, BATCH), jnp.float32),
        pltpu.VMEM((640,), jnp.float32),
        pltpu.VMEM_SHARED((N_PAD,), jnp.float32),
        pltpu.VMEM_SHARED((N_PAD,), jnp.float32),
    ],
)


def _coef_body(src_hbm, dst_hbm, ew_hbm, so_hbm, si_hbm, c_out,
               src_v, dst_v, ew_v, c_v, so_v, si_v):
    c = lax.axis_index("c")
    s = lax.axis_index("s")
    pltpu.sync_copy(src_hbm.at[c, s], src_v)
    pltpu.sync_copy(dst_hbm.at[c, s], dst_v)
    pltpu.sync_copy(ew_hbm.at[c, s], ew_v)
    pltpu.sync_copy(so_hbm, so_v)
    pltpu.sync_copy(si_hbm, si_v)

    def cbody(b, carry):
        for kk in range(BATCH // LANES):
            sl = pl.ds(kk * LANES, LANES)
            sv = plsc.load_gather(so_v, [src_v[b, sl]])
            dv = plsc.load_gather(si_v, [dst_v[b, sl]])
            c_v[b, sl] = ew_v[b, sl] * sv * dv
        return carry

    lax.fori_loop(0, NB, cbody, 0)
    pltpu.sync_copy(c_v, c_out.at[c, s])


_coef = pl.kernel(
    _coef_body,
    out_type=jax.ShapeDtypeStruct((NC, NS, NB, BATCH), jnp.float32),
    mesh=_MESH,
    compiler_params=_SC_PARAMS,
    scratch_types=[
        pltpu.VMEM((NB, BATCH), jnp.int32),
        pltpu.VMEM((NB, BATCH), jnp.int32),
        pltpu.VMEM((NB, BATCH), jnp.float32),
        pltpu.VMEM((NB, BATCH), jnp.float32),
        pltpu.VMEM((N_PAD,), jnp.float32),
        pltpu.VMEM((N_PAD,), jnp.float32),
    ],
)


def _bcast_lane(vec16, j):
    return lax.gather(
        vec16,
        jnp.full((LANES, 1), j, jnp.int32),
        lax.GatherDimensionNumbers(
            offset_dims=(), collapsed_slice_dims=(0,), start_index_map=(0,)
        ),
        (1,),
        mode=lax.GatherScatterMode.PROMISE_IN_BOUNDS,
    )


NB0 = 40
NB1 = 40
CHUNK = 40
TOTB = E_PAD // BATCH


def _spmm_body(d, nd, x_hbm, src_hbm, dst_hbm, c_hbm, out_hbm,
               src_v, dst_v, c_v, rows0, rows1, acc,
               gsem0, gsem1, ssem0, ssem1):
    c = lax.axis_index("c")
    s = lax.axis_index("s")

    z16 = jnp.zeros((LANES,), jnp.float32)

    def zrow(i, carry):
        for kk in range(d // LANES):
            rows0[i, pl.ds(kk * LANES, LANES)] = z16
        return carry

    lax.fori_loop(0, ZROWS, zrow, 0)
    for j in range(ROWS_PER_SUB // ZROWS):
        pltpu.sync_copy(rows0, acc.at[pl.ds(s * ROWS_PER_SUB + j * ZROWS, ZROWS)])
    plsc.subcore_barrier()

    def _gather(t, buf, sem):
        pltpu.async_copy(x_hbm.at[src_v.at[t]], buf, sem)

    def _wait_gather(t, buf, sem):
        pltpu.make_async_copy(x_hbm.at[src_v.at[t]], buf, sem).wait()

    def _mult(t, buf):
        @plsc.parallel_loop(0, BATCH // LANES, 1, unroll=2)
        def blk(k):
            w16 = c_v[t, pl.ds(k * LANES, LANES)]
            for j in range(LANES):
                w = _bcast_lane(w16, j)
                e = k * LANES + j
                for kk in range(nd):
                    sl = pl.ds(kk * LANES, LANES)
                    buf[e, sl] = buf[e, sl] * w

    def _run_chunk(base_b, nbc):
        sl_in = pl.ds(pl.multiple_of(base_b, 8), nbc)
        sl_v = pl.ds(0, nbc)
        pltpu.sync_copy(src_hbm.at[sl_in], src_v.at[sl_v])
        pltpu.sync_copy(dst_hbm.at[sl_in], dst_v.at[sl_v])
        pltpu.sync_copy(c_hbm.at[sl_in], c_v.at[sl_v])

        _gather(0, rows0, gsem0)
        _gather(1, rows1, gsem1)

        def mbody(k, carry):
            t0 = 2 * k
            t1 = 2 * k + 1
            _wait_gather(t0, rows0, gsem0)
            _mult(t0, rows0)
            pltpu.sync_copy(rows0, acc.at[dst_v.at[t0]], add=True)

            @pl.when(k < nbc // 2 - 1)
            def _():
                _gather(t0 + 2, rows0, gsem0)

            _wait_gather(t1, rows1, gsem1)
            _mult(t1, rows1)
            pltpu.sync_copy(rows1, acc.at[dst_v.at[t1]], add=True)

            @pl.when(k < nbc // 2 - 1)
            def _():
                _gather(t1 + 2, rows1, gsem1)

            return carry

        lax.fori_loop(0, nbc // 2, mbody, 0)

    @pl.when(c == 0)
    def _():
        _run_chunk(s * NB0, NB0)

    @pl.when(c == 1)
    def _():
        _run_chunk(NS * NB0 + s * NB1, NB1)

    plsc.subcore_barrier()
    for j in range(ROWS_PER_SUB // ZROWS):
        sl = pl.ds(s * ROWS_PER_SUB + j * ZROWS, ZROWS)
        pltpu.sync_copy(acc.at[sl], out_hbm.at[c, sl])


def _make_spmm(d, nd):
    return pl.kernel(
        functools.partial(_spmm_body, d, nd),
        out_type=jax.ShapeDtypeStruct((NC, N_PAD, d), jnp.float32),
        mesh=_MESH,
        compiler_params=_SC_PARAMS,
        scratch_types=[
            pltpu.VMEM((CHUNK, BATCH), jnp.int32),
            pltpu.VMEM((CHUNK, BATCH), jnp.int32),
            pltpu.VMEM((CHUNK, BATCH), jnp.float32),
            pltpu.VMEM((BATCH, d), jnp.float32),
            pltpu.VMEM((BATCH, d), jnp.float32),
            pltpu.VMEM_SHARED((N_PAD, d), jnp.float32),
            pltpu.SemaphoreType.DMA,
            pltpu.SemaphoreType.DMA,
            pltpu.SemaphoreType.DMA,
            pltpu.SemaphoreType.DMA,
        ],
    )


_spmm128 = _make_spmm(128, 8)
_spmm64 = _make_spmm(128, 4)


def _scales_body(o0_ref, o1_ref, i0_ref, i1_ref, so_ref, si_ref):
    dego = o0_ref[...] + o1_ref[...]
    degi = i0_ref[...] + i1_ref[...]
    so_ref[...] = lax.rsqrt(jnp.maximum(dego, 1.0))
    si_ref[...] = lax.rsqrt(jnp.maximum(degi, 1.0))


_scales = pl.pallas_call(
    _scales_body,
    out_shape=(
        jax.ShapeDtypeStruct((N_PAD,), jnp.float32),
        jax.ShapeDtypeStruct((N_PAD,), jnp.float32),
    ),
)

_ROWB = 2000
_GRID = N // _ROWB


def _mm_body(x_ref, w_ref, o_ref):
    o_ref[...] = jnp.dot(x_ref[...], w_ref[...], preferred_element_type=jnp.float32)


def _psum_bias_mm_body(z_ref, b_ref, w_ref, o_ref):
    h = z_ref[0] + z_ref[1] + b_ref[...]
    o_ref[...] = jnp.dot(h, w_ref[...], preferred_element_type=jnp.float32)


def _psum_bias_body(z_ref, b_ref, o_ref):
    o_ref[...] = z_ref[0] + z_ref[1] + b_ref[...]


def _psum_mm_bias_body(z_ref, w_ref, b_ref, o_ref):
    h = z_ref[0] + z_ref[1]
    o_ref[...] = jnp.dot(h, w_ref[...], preferred_element_type=jnp.float32) + b_ref[...]


def _x_spec(dcol):
    return pl.BlockSpec((_ROWB, dcol), lambda i: (i, 0))


def _z_spec(dcol):
    return pl.BlockSpec((NC, _ROWB, dcol), lambda i: (0, i, 0))


def _w_spec(din, dout):
    return pl.BlockSpec((din, dout), lambda i: (0, 0))


def _b_spec(dout):
    return pl.BlockSpec((1, dout), lambda i: (0, 0))


def _out_shape(dout):
    return jax.ShapeDtypeStruct((N, dout), jnp.float32)


_stage_a = pl.pallas_call(
    _mm_body, grid=(_GRID,),
    in_specs=[_x_spec(256), _w_spec(256, 128)],
    out_specs=_x_spec(128), out_shape=_out_shape(128),
)

_stage_b = pl.pallas_call(
    _psum_bias_mm_body, grid=(_GRID,),
    in_specs=[_z_spec(128), _b_spec(128), _w_spec(128, 128)],
    out_specs=_x_spec(128), out_shape=_out_shape(128),
)

_stage_c = pl.pallas_call(
    _psum_bias_body, grid=(_GRID,),
    in_specs=[_z_spec(128), _b_spec(128)],
    out_specs=_x_spec(128), out_shape=_out_shape(128),
)

_stage_d = pl.pallas_call(
    _psum_mm_bias_body, grid=(_GRID,),
    in_specs=[_z_spec(128), _w_spec(128, 128), _b_spec(128)],
    out_specs=_x_spec(128), out_shape=_out_shape(128),
)

_stage_e = pl.pallas_call(
    _psum_mm_bias_body, grid=(_GRID,),
    in_specs=[_z_spec(128), _w_spec(128, 256), _b_spec(256)],
    out_specs=_x_spec(256), out_shape=_out_shape(256),
)


def kernel(feat, edge_index, edge_weight, W0, b0, W1, b1, W2, b2, W3, b3):
    pad = E_PAD - E
    src = edge_index[0].astype(jnp.int32)
    dst = edge_index[1].astype(jnp.int32)
    pi = lax.iota(jnp.int32, pad)
    src_pad = pi % N
    dst_pad = N + pi % (N_PAD - N)
    zf = jnp.zeros((pad,), jnp.float32)
    shape4 = (NC, NS, NB, BATCH)
    srcp = jnp.concatenate([src, src_pad]).reshape(shape4)
    dstp = jnp.concatenate([dst, dst_pad]).reshape(shape4)
    ewp = jnp.concatenate([edge_weight.astype(jnp.float32), zf]).reshape(shape4)
    valp = jnp.concatenate([jnp.ones((E,), jnp.float32), zf]).reshape(shape4)

    deg_o0, deg_o1, deg_i0, deg_i1 = _deg(srcp, dstp, valp)
    s_out, s_in = _scales(deg_o0, deg_o1, deg_i0, deg_i1)
    cedge = _coef(srcp, dstp, ewp, s_out, s_in)

    srcf = srcp.reshape(TOTB, BATCH)
    dstf = dstp.reshape(TOTB, BATCH)
    cf = cedge.reshape(TOTB, BATCH)

    w1p = jnp.concatenate([W1, jnp.zeros((128, 64), jnp.float32)], axis=1)
    b1p = jnp.concatenate([b1, jnp.zeros((64,), jnp.float32)])
    w2p = jnp.concatenate([W2, jnp.zeros((64, 128), jnp.float32)], axis=0)

    x0 = _stage_a(feat, W0)
    z0 = _spmm128(x0, srcf, dstf, cf)
    x1 = _stage_b(z0, b0.reshape(1, -1), w1p)
    z1 = _spmm64(x1, srcf, dstf, cf)
    x2 = _stage_c(z1, b1p.reshape(1, -1))
    z2 = _spmm64(x2, srcf, dstf, cf)
    x3 = _stage_d(z2, w2p, b2.reshape(1, -1))
    z3 = _spmm128(x3, srcf, dstf, cf)
    return _stage_e(z3, W3, b3.reshape(1, -1))


# speedup vs baseline: 1.0655x; 1.0029x over previous
import functools

import jax
import jax.numpy as jnp
from jax import lax
from jax.experimental import pallas as pl
from jax.experimental.pallas import tpu as pltpu
from jax.experimental.pallas import tpu_sc as plsc

N = 10000
E = 160000
NC = 2
NS = 16
LANES = 16
BATCH = 128
NB = 40
E_PAD = NC * NS * NB * BATCH
N_PAD = 10240
ROWS_PER_SUB = N_PAD // NS
ZROWS = 128

_MESH = plsc.VectorSubcoreMesh(
    core_axis_name="c", subcore_axis_name="s", num_cores=NC, num_subcores=NS
)
_SC_PARAMS = pltpu.CompilerParams(needs_layout_passes=False)


def _deg_body(src_hbm, dst_hbm, val_hbm, out_o0, out_o1, out_i0, out_i1,
              src_v, dst_v, val_v, zero_v, acc_o, acc_i):
    c = lax.axis_index("c")
    s = lax.axis_index("s")
    pltpu.sync_copy(src_hbm.at[c, s], src_v)
    pltpu.sync_copy(dst_hbm.at[c, s], dst_v)
    pltpu.sync_copy(val_hbm.at[c, s], val_v)

    z16 = jnp.zeros((LANES,), jnp.float32)

    def zbody(k, carry):
        zero_v[pl.ds(k * LANES, LANES)] = z16
        return carry

    lax.fori_loop(0, 640 // LANES, zbody, 0)
    stripe = pl.ds(s * 640, 640)
    pltpu.sync_copy(zero_v, acc_o.at[stripe])
    pltpu.sync_copy(zero_v, acc_i.at[stripe])
    plsc.subcore_barrier()

    def body(b, carry):
        pltpu.sync_copy(val_v.at[b], acc_o.at[src_v.at[b]], add=True)
        pltpu.sync_copy(val_v.at[b], acc_i.at[dst_v.at[b]], add=True)
        return carry

    lax.fori_loop(0, NB, body, 0)
    plsc.subcore_barrier()

    @pl.when(c == 0)
    def _():
        pltpu.sync_copy(acc_o.at[stripe], out_o0.at[stripe])
        pltpu.sync_copy(acc_i.at[stripe], out_i0.at[stripe])

    @pl.when(c == 1)
    def _():
        pltpu.sync_copy(acc_o.at[stripe], out_o1.at[stripe])
        pltpu.sync_copy(acc_i.at[stripe], out_i1.at[stripe])


_deg = pl.kernel(
    _deg_body,
    out_type=tuple(jax.ShapeDtypeStruct((N_PAD,), jnp.float32) for _ in range(4)),
    mesh=_MESH,
    compiler_params=_SC_PARAMS,
    scratch_types=[
        pltpu.VMEM((NB, BATCH), jnp.int32),
        pltpu.VMEM((NB, BATCH), jnp.int32),
        pltpu.VMEM((NB, BATCH), jnp.float32),
        pltpu.VMEM((640,), jnp.float32),
        pltpu.VMEM_SHARED((N_PAD,), jnp.float32),
        pltpu.VMEM_SHARED((N_PAD,), jnp.float32),
    ],
)


def _coef_body(src_hbm, dst_hbm, ew_hbm, so_hbm, si_hbm, c_out,
               src_v, dst_v, ew_v, c_v, so_v, si_v):
    c = lax.axis_index("c")
    s = lax.axis_index("s")
    pltpu.sync_copy(src_hbm.at[c, s], src_v)
    pltpu.sync_copy(dst_hbm.at[c, s], dst_v)
    pltpu.sync_copy(ew_hbm.at[c, s], ew_v)
    pltpu.sync_copy(so_hbm, so_v)
    pltpu.sync_copy(si_hbm, si_v)

    def cbody(b, carry):
        for kk in range(BATCH // LANES):
            sl = pl.ds(kk * LANES, LANES)
            sv = plsc.load_gather(so_v, [src_v[b, sl]])
            dv = plsc.load_gather(si_v, [dst_v[b, sl]])
            c_v[b, sl] = ew_v[b, sl] * sv * dv
        return carry

    lax.fori_loop(0, NB, cbody, 0)
    pltpu.sync_copy(c_v, c_out.at[c, s])


_coef = pl.kernel(
    _coef_body,
    out_type=jax.ShapeDtypeStruct((NC, NS, NB, BATCH), jnp.float32),
    mesh=_MESH,
    compiler_params=_SC_PARAMS,
    scratch_types=[
        pltpu.VMEM((NB, BATCH), jnp.int32),
        pltpu.VMEM((NB, BATCH), jnp.int32),
        pltpu.VMEM((NB, BATCH), jnp.float32),
        pltpu.VMEM((NB, BATCH), jnp.float32),
        pltpu.VMEM((N_PAD,), jnp.float32),
        pltpu.VMEM((N_PAD,), jnp.float32),
    ],
)


def _bcast_lane(vec16, j):
    return lax.gather(
        vec16,
        jnp.full((LANES, 1), j, jnp.int32),
        lax.GatherDimensionNumbers(
            offset_dims=(), collapsed_slice_dims=(0,), start_index_map=(0,)
        ),
        (1,),
        mode=lax.GatherScatterMode.PROMISE_IN_BOUNDS,
    )


NB0 = 40
NB1 = 40
CHUNK = 40
TOTB = E_PAD // BATCH


def _spmm_body(d, nd, x_hbm, src_hbm, dst_hbm, c_hbm, out_hbm,
               src_v, dst_v, c_v, rows0, rows1, acc, gsem0, gsem1):
    c = lax.axis_index("c")
    s = lax.axis_index("s")

    z16 = jnp.zeros((LANES,), jnp.float32)

    def zrow(i, carry):
        for kk in range(d // LANES):
            rows0[i, pl.ds(kk * LANES, LANES)] = z16
        return carry

    lax.fori_loop(0, ZROWS, zrow, 0)
    for j in range(ROWS_PER_SUB // ZROWS):
        pltpu.sync_copy(rows0, acc.at[pl.ds(s * ROWS_PER_SUB + j * ZROWS, ZROWS)])
    plsc.subcore_barrier()

    def _gather(t, buf, sem):
        pltpu.async_copy(x_hbm.at[src_v.at[t]], buf, sem)

    def _wait_gather(t, buf, sem):
        pltpu.make_async_copy(x_hbm.at[src_v.at[t]], buf, sem).wait()

    def _mult(t, buf):
        @plsc.parallel_loop(0, BATCH // LANES, 1, unroll=2)
        def blk(k):
            w16 = c_v[t, pl.ds(k * LANES, LANES)]
            for j in range(LANES):
                w = _bcast_lane(w16, j)
                e = k * LANES + j
                for kk in range(nd):
                    sl = pl.ds(kk * LANES, LANES)
                    buf[e, sl] = buf[e, sl] * w

    def _run_chunk(base_b, nbc):
        sl_in = pl.ds(pl.multiple_of(base_b, 8), nbc)
        sl_v = pl.ds(0, nbc)
        pltpu.sync_copy(src_hbm.at[sl_in], src_v.at[sl_v])
        pltpu.sync_copy(dst_hbm.at[sl_in], dst_v.at[sl_v])
        pltpu.sync_copy(c_hbm.at[sl_in], c_v.at[sl_v])

        _gather(0, rows0, gsem0)
        _gather(1, rows1, gsem1)

        def mbody(k, carry):
            t0 = 2 * k
            t1 = 2 * k + 1
            _wait_gather(t0, rows0, gsem0)
            _mult(t0, rows0)
            pltpu.sync_copy(rows0, acc.at[dst_v.at[t0]], add=True)

            @pl.when(k < nbc // 2 - 1)
            def _():
                _gather(t0 + 2, rows0, gsem0)

            _wait_gather(t1, rows1, gsem1)
            _mult(t1, rows1)
            pltpu.sync_copy(rows1, acc.at[dst_v.at[t1]], add=True)

            @pl.when(k < nbc // 2 - 1)
            def _():
                _gather(t1 + 2, rows1, gsem1)

            return carry

        lax.fori_loop(0, nbc // 2, mbody, 0)

    @pl.when(c == 0)
    def _():
        _run_chunk(s * NB0, NB0)

    @pl.when(c == 1)
    def _():
        _run_chunk(NS * NB0 + s * NB1, NB1)

    plsc.subcore_barrier()
    for j in range(ROWS_PER_SUB // ZROWS):
        sl = pl.ds(s * ROWS_PER_SUB + j * ZROWS, ZROWS)
        pltpu.sync_copy(acc.at[sl], out_hbm.at[c, sl])


def _make_spmm(d, nd):
    return pl.kernel(
        functools.partial(_spmm_body, d, nd),
        out_type=jax.ShapeDtypeStruct((NC, N_PAD, d), jnp.float32),
        mesh=_MESH,
        compiler_params=_SC_PARAMS,
        scratch_types=[
            pltpu.VMEM((CHUNK, BATCH), jnp.int32),
            pltpu.VMEM((CHUNK, BATCH), jnp.int32),
            pltpu.VMEM((CHUNK, BATCH), jnp.float32),
            pltpu.VMEM((BATCH, d), jnp.float32),
            pltpu.VMEM((BATCH, d), jnp.float32),
            pltpu.VMEM_SHARED((N_PAD, d), jnp.float32),
            pltpu.SemaphoreType.DMA,
            pltpu.SemaphoreType.DMA,
        ],
    )


_spmm128 = _make_spmm(128, 8)
_spmm64 = _make_spmm(128, 4)


def _scales_body(o0_ref, o1_ref, i0_ref, i1_ref, so_ref, si_ref):
    dego = o0_ref[...] + o1_ref[...]
    degi = i0_ref[...] + i1_ref[...]
    so_ref[...] = lax.rsqrt(jnp.maximum(dego, 1.0))
    si_ref[...] = lax.rsqrt(jnp.maximum(degi, 1.0))


_scales = pl.pallas_call(
    _scales_body,
    out_shape=(
        jax.ShapeDtypeStruct((N_PAD,), jnp.float32),
        jax.ShapeDtypeStruct((N_PAD,), jnp.float32),
    ),
)

_ROWB = 2000
_GRID = N // _ROWB


def _mm_body(x_ref, w_ref, o_ref):
    o_ref[...] = jnp.dot(x_ref[...], w_ref[...], preferred_element_type=jnp.float32)


def _psum_bias_mm_body(z_ref, b_ref, w_ref, o_ref):
    h = z_ref[0] + z_ref[1] + b_ref[...]
    o_ref[...] = jnp.dot(h, w_ref[...], preferred_element_type=jnp.float32)


def _psum_bias_body(z_ref, b_ref, o_ref):
    o_ref[...] = z_ref[0] + z_ref[1] + b_ref[...]


def _psum_mm_bias_body(z_ref, w_ref, b_ref, o_ref):
    h = z_ref[0] + z_ref[1]
    o_ref[...] = jnp.dot(h, w_ref[...], preferred_element_type=jnp.float32) + b_ref[...]


def _x_spec(dcol):
    return pl.BlockSpec((_ROWB, dcol), lambda i: (i, 0))


def _z_spec(dcol):
    return pl.BlockSpec((NC, _ROWB, dcol), lambda i: (0, i, 0))


def _w_spec(din, dout):
    return pl.BlockSpec((din, dout), lambda i: (0, 0))


def _b_spec(dout):
    return pl.BlockSpec((1, dout), lambda i: (0, 0))


def _out_shape(dout):
    return jax.ShapeDtypeStruct((N, dout), jnp.float32)


_stage_a = pl.pallas_call(
    _mm_body, grid=(_GRID,),
    in_specs=[_x_spec(256), _w_spec(256, 128)],
    out_specs=_x_spec(128), out_shape=_out_shape(128),
)

_stage_b = pl.pallas_call(
    _psum_bias_mm_body, grid=(_GRID,),
    in_specs=[_z_spec(128), _b_spec(128), _w_spec(128, 128)],
    out_specs=_x_spec(128), out_shape=_out_shape(128),
)

_stage_c = pl.pallas_call(
    _psum_bias_body, grid=(_GRID,),
    in_specs=[_z_spec(128), _b_spec(128)],
    out_specs=_x_spec(128), out_shape=_out_shape(128),
)

_stage_d = pl.pallas_call(
    _psum_mm_bias_body, grid=(_GRID,),
    in_specs=[_z_spec(128), _w_spec(128, 128), _b_spec(128)],
    out_specs=_x_spec(128), out_shape=_out_shape(128),
)

_stage_e = pl.pallas_call(
    _psum_mm_bias_body, grid=(_GRID,),
    in_specs=[_z_spec(128), _w_spec(128, 256), _b_spec(256)],
    out_specs=_x_spec(256), out_shape=_out_shape(256),
)


def kernel(feat, edge_index, edge_weight, W0, b0, W1, b1, W2, b2, W3, b3):
    pad = E_PAD - E
    src = edge_index[0].astype(jnp.int32)
    dst = edge_index[1].astype(jnp.int32)
    pi = lax.iota(jnp.int32, pad)
    src_pad = pi % N
    dst_pad = N + pi % (N_PAD - N)
    zf = jnp.zeros((pad,), jnp.float32)
    shape4 = (NC, NS, NB, BATCH)
    srcp = jnp.concatenate([src, src_pad]).reshape(shape4)
    dstp = jnp.concatenate([dst, dst_pad]).reshape(shape4)
    ewp = jnp.concatenate([edge_weight.astype(jnp.float32), zf]).reshape(shape4)
    valp = jnp.concatenate([jnp.ones((E,), jnp.float32), zf]).reshape(shape4)

    deg_o0, deg_o1, deg_i0, deg_i1 = _deg(srcp, dstp, valp)
    s_out, s_in = _scales(deg_o0, deg_o1, deg_i0, deg_i1)
    cedge = _coef(srcp, dstp, ewp, s_out, s_in)

    srcf = srcp.reshape(TOTB, BATCH)
    dstf = dstp.reshape(TOTB, BATCH)
    cf = cedge.reshape(TOTB, BATCH)

    w1p = jnp.concatenate([W1, jnp.zeros((128, 64), jnp.float32)], axis=1)
    b1p = jnp.concatenate([b1, jnp.zeros((64,), jnp.float32)])
    w2p = jnp.concatenate([W2, jnp.zeros((64, 128), jnp.float32)], axis=0)

    x0 = _stage_a(feat, W0)
    z0 = _spmm128(x0, srcf, dstf, cf)
    x1 = _stage_b(z0, b0.reshape(1, -1), w1p)
    z1 = _spmm64(x1, srcf, dstf, cf)
    x2 = _stage_c(z1, b1p.reshape(1, -1))
    z2 = _spmm64(x2, srcf, dstf, cf)
    x3 = _stage_d(z2, w2p, b2.reshape(1, -1))
    z3 = _spmm128(x3, srcf, dstf, cf)
    return _stage_e(z3, W3, b3.reshape(1, -1))
